# baseline (speedup 1.0000x reference)
import functools

import jax
import jax.numpy as jnp
import numpy as np
from jax import lax
from jax.experimental import pallas as pl
from jax.experimental.pallas import tpu as pltpu
from jax.experimental.pallas import tpu_sc as plsc

K = 6
NC_PAD = 10240
CHUNKS = 5
NF_CHUNK = 8000
NF_CPAD = 8192
BF = 320
RB = 400
FB = 160
CH = 192
SC_WORKERS = 32


def _bf16_bits(v):
    return lax.shift_right_logical(
        lax.bitcast_convert_type(v, jnp.int32) + jnp.int32(0x8000), 16)


def _y_body(x_ref, wt_ref, b_ref, y_ref):
    y = jnp.dot(
        x_ref[...], wt_ref[...],
        preferred_element_type=jnp.float32,
        precision=lax.Precision.HIGHEST,
    ) + b_ref[0:1, :]
    h = y.shape[1] // 2
    y_ref[...] = lax.shift_left(_bf16_bits(y[:, h:]), 16) | _bf16_bits(y[:, :h])


def _tc_y(x, wt, b8):
    n = x.shape[0]
    d = wt.shape[1]
    return pl.pallas_call(
        _y_body,
        grid=(n // RB,),
        in_specs=[
            pl.BlockSpec((RB, x.shape[1]), lambda i: (i, 0)),
            pl.BlockSpec((wt.shape[0], d), lambda i: (0, 0)),
            pl.BlockSpec((8, d), lambda i: (0, 0)),
        ],
        out_specs=pl.BlockSpec((RB, d // 2), lambda i: (i, 0)),
        out_shape=jax.ShapeDtypeStruct((n, d // 2), jnp.int32),
    )(x, wt, b8)


FOLD = 16
CPF = NC_PAD // FOLD
BIG_KEY = int(np.float32(1e30).view(np.int32))


def _knn_body(pf_ref, pct_ref, idx_ref, wn_ref):
    pf = pf_ref[...]
    pct = pct_ref[...]

    cross = jnp.dot(pf, pct, preferred_element_type=jnp.float32,
                    precision=lax.Precision.HIGHEST)
    pf2 = jnp.sum(pf * pf, axis=1, keepdims=True)
    pc2 = jnp.sum(pct * pct, axis=0, keepdims=True)

    f1 = None
    f2 = jnp.full((BF, CPF), BIG_KEY, jnp.int32)
    for f in range(FOLD):
        sl = slice(f * CPF, (f + 1) * CPF)
        d2f = (pf2 + pc2[:, sl]) - 2.0 * cross[:, sl]
        key = (lax.bitcast_convert_type(d2f, jnp.int32) & ~15) | f
        if f == 0:
            f1 = key
        else:
            f2 = jnp.minimum(f2, jnp.maximum(key, f1))
            f1 = jnp.minimum(f1, key)

    col = lax.broadcasted_iota(jnp.int32, (BF, CPF), 1)
    cols_k = lax.broadcasted_iota(jnp.int32, (BF, K), 1)
    acc_idx = jnp.zeros((BF, K), jnp.int32)
    acc_w = jnp.zeros((BF, K), jnp.float32)
    wsum = None
    bigi = jnp.int32(2 ** 30)
    for k in range(K):
        mk = jnp.min(f1, axis=1, keepdims=True)
        ck = jnp.min(jnp.where(f1 == mk, col, bigi), axis=1, keepdims=True)
        val = lax.bitcast_convert_type(mk & ~15, jnp.float32)
        idxk = (mk & 15) * CPF + ck
        wk = 1.0 / jnp.maximum(val, 1e-16)
        acc_idx = jnp.where(cols_k == k, idxk, acc_idx)
        acc_w = jnp.where(cols_k == k, wk, acc_w)
        wsum = wk if k == 0 else wsum + wk
        if k + 1 < K:
            hit = col == ck
            f1 = jnp.where(hit, f2, f1)
            f2 = jnp.where(hit, BIG_KEY, f2)
    idx_ref[...] = acc_idx
    wn_ref[...] = acc_w / wsum


def _tc_knn(pf, pct):
    n = pf.shape[0]
    return pl.pallas_call(
        _knn_body,
        grid=(n // BF,),
        in_specs=[
            pl.BlockSpec((BF, 3), lambda i: (i, 0)),
            pl.BlockSpec((3, NC_PAD), lambda i: (0, 0)),
        ],
        out_specs=[
            pl.BlockSpec((BF, K), lambda i: (i, 0)),
            pl.BlockSpec((BF, K), lambda i: (i, 0)),
        ],
        out_shape=[
            jax.ShapeDtypeStruct((n, K), jnp.int32),
            jax.ShapeDtypeStruct((n, K), jnp.float32),
        ],
    )(pf, pct)


def _sc_gather(y, idx_flat):
    d = y.shape[1]
    total = idx_flat.shape[0]
    per_w = total // SC_WORKERS
    n_items = per_w // CH
    mesh = plsc.VectorSubcoreMesh(core_axis_name="c", subcore_axis_name="s")

    @functools.partial(
        pl.kernel,
        mesh=mesh,
        out_type=jax.ShapeDtypeStruct((total, d), jnp.int32),
        scratch_types=[
            pltpu.VMEM((per_w,), jnp.int32),
            pltpu.VMEM((CH, d), jnp.int32),
            pltpu.VMEM((CH, d), jnp.int32),
            pltpu.SemaphoreType.DMA,
            pltpu.SemaphoreType.DMA,
        ],
    )
    def gather_kernel(y_hbm, i_hbm, o_hbm, idx_v, rows0, rows1, sem0, sem1):
        wid = lax.axis_index("s") * 2 + lax.axis_index("c")
        base = wid * per_w
        pltpu.sync_copy(i_hbm.at[pl.ds(base, per_w)], idx_v)
        pltpu.async_copy(y_hbm.at[idx_v.at[pl.ds(0, CH)]], rows0, sem0)
        pltpu.async_copy(y_hbm.at[idx_v.at[pl.ds(CH, CH)]], rows1, sem1)

        @pl.loop(0, (n_items - 2) * CH, step=2 * CH)
        def _(off):
            for b, (rows, sem) in enumerate(((rows0, sem0), (rows1, sem1))):
                o = off + b * CH
                pltpu.make_async_copy(
                    y_hbm.at[idx_v.at[pl.ds(o, CH)]], rows, sem).wait()
                pltpu.sync_copy(rows, o_hbm.at[pl.ds(base + o, CH)])
                pltpu.async_copy(
                    y_hbm.at[idx_v.at[pl.ds(o + 2 * CH, CH)]], rows, sem)

        for b, (rows, sem) in enumerate(((rows0, sem0), (rows1, sem1))):
            o = (n_items - 2 + b) * CH
            pltpu.make_async_copy(
                y_hbm.at[idx_v.at[pl.ds(o, CH)]], rows, sem).wait()
            pltpu.sync_copy(rows, o_hbm.at[pl.ds(base + o, CH)])

    return gather_kernel(y, idx_flat)


def _combine_body(g_ref, wn_ref, out_ref):
    w = wn_ref[...]
    acc_lo = acc_hi = None
    for k in range(K):
        p = g_ref[k]
        lo = lax.bitcast_convert_type(lax.shift_left(p, 16), jnp.float32)
        hi = lax.bitcast_convert_type(p & jnp.int32(-65536), jnp.float32)
        wk = w[:, k:k + 1]
        if k == 0:
            acc_lo, acc_hi = lo * wk, hi * wk
        else:
            acc_lo, acc_hi = acc_lo + lo * wk, acc_hi + hi * wk
    h = acc_lo.shape[1]
    out_ref[:, :h] = acc_lo
    out_ref[:, h:] = acc_hi


def _tc_combine(g, wn):
    n = wn.shape[0]
    dp = g.shape[2]
    return pl.pallas_call(
        _combine_body,
        grid=(n // FB,),
        in_specs=[
            pl.BlockSpec((K, FB, dp), lambda i: (0, i, 0)),
            pl.BlockSpec((FB, K), lambda i: (i, 0)),
        ],
        out_specs=pl.BlockSpec((FB, 2 * dp), lambda i: (i, 0)),
        out_shape=jax.ShapeDtypeStruct((n, 2 * dp), jnp.float32),
    )(g, wn)


def kernel(x_coarse, pos_coarse, pos_fine, W, b):
    nc = pos_coarse.shape[0]
    d_out = W.shape[0]

    y = _tc_y(x_coarse, W.T, jnp.tile(b[None, :], (8, 1)))

    pct = jnp.concatenate(
        [pos_coarse.T,
         jnp.full((3, NC_PAD - nc), 1e3, jnp.float32)], axis=1)

    outs = []
    for c in range(CHUNKS):
        pf_c = lax.slice_in_dim(pos_fine, c * NF_CHUNK, (c + 1) * NF_CHUNK)
        idx, wn = _tc_knn(pf_c, pct)
        idx_km = jnp.pad(idx, ((0, NF_CPAD - NF_CHUNK), (0, 0))).T.reshape(-1)
        g = _sc_gather(y, idx_km).reshape(K, NF_CPAD, d_out // 2)
        outs.append(_tc_combine(g, wn))
    return jnp.concatenate(outs, axis=0)


# speedup vs baseline: 1.7218x; 1.7218x over previous
import functools

import jax
import jax.numpy as jnp
import numpy as np
from jax import lax
from jax.experimental import pallas as pl
from jax.experimental.pallas import tpu as pltpu
from jax.experimental.pallas import tpu_sc as plsc

K = 6
NC_PAD = 10240
CHUNKS = 5
BF = 400
RB = 400
FB = 160
CH = 192
SC_WORKERS = 32


def _bf16_bits(v):
    return lax.shift_right_logical(
        lax.bitcast_convert_type(v, jnp.int32) + jnp.int32(0x8000), 16)


def _y_body(x_ref, wt_ref, b_ref, y_ref):
    y = jnp.dot(
        x_ref[...], wt_ref[...],
        preferred_element_type=jnp.float32,
        precision=lax.Precision.HIGHEST,
    ) + b_ref[0:1, :]
    h = y.shape[1] // 2
    y_ref[...] = lax.shift_left(_bf16_bits(y[:, h:]), 16) | _bf16_bits(y[:, :h])


def _tc_y(x, wt, b8):
    n = x.shape[0]
    d = wt.shape[1]
    return pl.pallas_call(
        _y_body,
        grid=(n // RB,),
        in_specs=[
            pl.BlockSpec((RB, x.shape[1]), lambda i: (i, 0)),
            pl.BlockSpec((wt.shape[0], d), lambda i: (0, 0)),
            pl.BlockSpec((8, d), lambda i: (0, 0)),
        ],
        out_specs=pl.BlockSpec((RB, d // 2), lambda i: (i, 0)),
        out_shape=jax.ShapeDtypeStruct((n, d // 2), jnp.int32),
    )(x, wt, b8)


FOLD = 16
CPF = NC_PAD // FOLD
BIG_KEY = int(np.float32(1e30).view(np.int32))


def _knn_body(pf_ref, pct_ref, idx_ref, wn_ref):
    pf = pf_ref[...]

    f1 = None
    f2 = jnp.full((BF, CPF), BIG_KEY, jnp.int32)
    for f in range(FOLD):
        d2f = None
        for d in range(3):
            diff = pf[:, d:d + 1] - pct_ref[d:d + 1, f * CPF:(f + 1) * CPF]
            d2f = diff * diff if d == 0 else d2f + diff * diff
        key = (lax.bitcast_convert_type(d2f, jnp.int32) & ~15) | f
        if f == 0:
            f1 = key
        else:
            f2 = jnp.minimum(f2, jnp.maximum(key, f1))
            f1 = jnp.minimum(f1, key)

    col = lax.broadcasted_iota(jnp.int32, (BF, CPF), 1)
    cols_k = lax.broadcasted_iota(jnp.int32, (BF, K), 1)
    acc_idx = jnp.zeros((BF, K), jnp.int32)
    acc_w = jnp.zeros((BF, K), jnp.float32)
    wsum = None
    bigi = jnp.int32(2 ** 30)
    for k in range(K):
        mk = jnp.min(f1, axis=1, keepdims=True)
        ck = jnp.min(jnp.where(f1 == mk, col, bigi), axis=1, keepdims=True)
        val = lax.bitcast_convert_type(mk & ~15, jnp.float32)
        idxk = (mk & 15) * CPF + ck
        wk = 1.0 / jnp.maximum(val, 1e-16)
        acc_idx = jnp.where(cols_k == k, idxk, acc_idx)
        acc_w = jnp.where(cols_k == k, wk, acc_w)
        wsum = wk if k == 0 else wsum + wk
        if k + 1 < K:
            hit = col == ck
            f1 = jnp.where(hit, f2, f1)
            f2 = jnp.where(hit, BIG_KEY, f2)
    idx_ref[...] = acc_idx
    wn_ref[...] = acc_w / wsum


def _tc_knn(pf, pct):
    n = pf.shape[0]
    return pl.pallas_call(
        _knn_body,
        grid=(n // BF,),
        in_specs=[
            pl.BlockSpec((BF, 3), lambda i: (i, 0)),
            pl.BlockSpec((3, NC_PAD), lambda i: (0, 0)),
        ],
        out_specs=[
            pl.BlockSpec((BF, K), lambda i: (i, 0)),
            pl.BlockSpec((BF, K), lambda i: (i, 0)),
        ],
        out_shape=[
            jax.ShapeDtypeStruct((n, K), jnp.int32),
            jax.ShapeDtypeStruct((n, K), jnp.float32),
        ],
    )(pf, pct)


def _sc_gather(y, idx_flat):
    d = y.shape[1]
    total = idx_flat.shape[0]
    per_w = total // SC_WORKERS
    n_items = per_w // CH
    mesh = plsc.VectorSubcoreMesh(core_axis_name="c", subcore_axis_name="s")

    @functools.partial(
        pl.kernel,
        mesh=mesh,
        out_type=jax.ShapeDtypeStruct((total, d), jnp.int32),
        scratch_types=[
            pltpu.VMEM((per_w,), jnp.int32),
            pltpu.VMEM((CH, d), jnp.int32),
            pltpu.VMEM((CH, d), jnp.int32),
            pltpu.SemaphoreType.DMA,
            pltpu.SemaphoreType.DMA,
        ],
    )
    def gather_kernel(y_hbm, i_hbm, o_hbm, idx_v, rows0, rows1, sem0, sem1):
        wid = lax.axis_index("s") * 2 + lax.axis_index("c")
        base = wid * per_w
        pltpu.sync_copy(i_hbm.at[pl.ds(base, per_w)], idx_v)
        pltpu.async_copy(y_hbm.at[idx_v.at[pl.ds(0, CH)]], rows0, sem0)
        pltpu.async_copy(y_hbm.at[idx_v.at[pl.ds(CH, CH)]], rows1, sem1)

        @pl.loop(0, (n_items - 2) * CH, step=2 * CH)
        def _(off):
            for b, (rows, sem) in enumerate(((rows0, sem0), (rows1, sem1))):
                o = off + b * CH
                pltpu.make_async_copy(
                    y_hbm.at[idx_v.at[pl.ds(o, CH)]], rows, sem).wait()
                pltpu.sync_copy(rows, o_hbm.at[pl.ds(base + o, CH)])
                pltpu.async_copy(
                    y_hbm.at[idx_v.at[pl.ds(o + 2 * CH, CH)]], rows, sem)

        for b, (rows, sem) in enumerate(((rows0, sem0), (rows1, sem1))):
            o = (n_items - 2 + b) * CH
            pltpu.make_async_copy(
                y_hbm.at[idx_v.at[pl.ds(o, CH)]], rows, sem).wait()
            pltpu.sync_copy(rows, o_hbm.at[pl.ds(base + o, CH)])

    return gather_kernel(y, idx_flat)


def _combine_body(g_ref, wn_ref, out_ref):
    w = wn_ref[...]
    acc_lo = acc_hi = None
    for k in range(K):
        p = g_ref[k]
        lo = lax.bitcast_convert_type(lax.shift_left(p, 16), jnp.float32)
        hi = lax.bitcast_convert_type(p & jnp.int32(-65536), jnp.float32)
        wk = w[:, k:k + 1]
        if k == 0:
            acc_lo, acc_hi = lo * wk, hi * wk
        else:
            acc_lo, acc_hi = acc_lo + lo * wk, acc_hi + hi * wk
    h = acc_lo.shape[1]
    out_ref[:, :h] = acc_lo
    out_ref[:, h:] = acc_hi


def _tc_combine(g, wn):
    n = wn.shape[0]
    dp = g.shape[2]
    return pl.pallas_call(
        _combine_body,
        grid=(n // FB,),
        in_specs=[
            pl.BlockSpec((K, FB, dp), lambda i: (0, i, 0)),
            pl.BlockSpec((FB, K), lambda i: (i, 0)),
        ],
        out_specs=pl.BlockSpec((FB, 2 * dp), lambda i: (i, 0)),
        out_shape=jax.ShapeDtypeStruct((n, 2 * dp), jnp.float32),
    )(g, wn)


def _pipeline(x_coarse, pos_coarse, pos_fine, W, b):
    nc = pos_coarse.shape[0]
    d_out = W.shape[0]
    nf = pos_fine.shape[0]
    nf_chunk = nf // CHUNKS
    nf_cpad = -(-nf_chunk // 1024) * 1024

    y = _tc_y(x_coarse, W.T, jnp.tile(b[None, :], (8, 1)))

    pct = jnp.concatenate(
        [pos_coarse.T,
         jnp.full((3, NC_PAD - nc), 1e3, jnp.float32)], axis=1)

    outs = []
    for c in range(CHUNKS):
        pf_c = lax.slice_in_dim(pos_fine, c * nf_chunk, (c + 1) * nf_chunk)
        idx, wn = _tc_knn(pf_c, pct)
        idx_km = jnp.pad(idx, ((0, nf_cpad - nf_chunk), (0, 0))).T.reshape(-1)
        g = _sc_gather(y, idx_km).reshape(K, nf_cpad, d_out // 2)
        outs.append(_tc_combine(g, wn))
    return jnp.concatenate(outs, axis=0)


def kernel(x_coarse, pos_coarse, pos_fine, W, b):
    devs = jax.devices()
    if len(devs) < 2 or pos_fine.shape[0] % (2 * CHUNKS * BF) != 0:
        return _pipeline(x_coarse, pos_coarse, pos_fine, W, b)
    mesh = jax.sharding.Mesh(np.array(devs[:2]), ("d",))
    p = jax.sharding.PartitionSpec
    return jax.shard_map(
        _pipeline, mesh=mesh,
        in_specs=(p(), p(), p("d"), p(), p()),
        out_specs=p("d"), check_vma=False,
    )(x_coarse, pos_coarse, pos_fine, W, b)
